# baseline (speedup 1.0000x reference)
import functools

import jax
import jax.numpy as jnp
from jax import lax
from jax.experimental import pallas as pl
from jax.experimental.pallas import tpu as pltpu
from jax.experimental.pallas import tpu_sc as plsc

N = 10000
E = 320000
D = 128
CW = 128
NC, NS = 2, 16
NW = NC * NS
EPW = E // NW
C = 80
NCH = EPW // C
RPT = 624
RPT_LAST = N - RPT * (NS - 1)


def _wb(src_sh, out_h, c, sbase, s):
    def wb(sz):
        pltpu.sync_copy(src_sh.at[pl.ds(sbase, sz)],
                        out_h.at[c, pl.ds(sbase, sz)])
    pl.when(s < NS - 1)(lambda: wb(RPT))
    pl.when(s == NS - 1)(lambda: wb(RPT_LAST))


def _sc_sum_body(x_h, src_h, dst_h, zeros_h, out_h,
                 src_all, dst_all, rows_v0, rows_v1, acc_sh,
                 sem0, sem1):
    c = lax.axis_index("c")
    s = lax.axis_index("s")
    wid = s * NC + c
    sbase = s * RPT

    def zstripe(sz):
        pltpu.sync_copy(zeros_h.at[pl.ds(0, sz)],
                        acc_sh.at[pl.ds(sbase, sz)])
    pl.when(s < NS - 1)(lambda: zstripe(RPT))
    pl.when(s == NS - 1)(lambda: zstripe(RPT_LAST))
    pltpu.sync_copy(src_h.at[wid], src_all)
    pltpu.sync_copy(dst_h.at[wid], dst_all)
    plsc.subcore_barrier()

    bufs = ((rows_v0, sem0), (rows_v1, sem1))

    def fire(g, b):
        rows_v, sem = bufs[b]
        idx = src_all.at[pl.ds(g * C, C)]
        pltpu.make_async_copy(x_h.at[idx], rows_v, sem).start()

    def finish(g, b):
        rows_v, sem = bufs[b]
        idx = src_all.at[pl.ds(g * C, C)]
        pltpu.make_async_copy(x_h.at[idx], rows_v, sem).wait()
        pltpu.sync_copy(rows_v, acc_sh.at[dst_all.at[g]], add=True)

    fire(0, 0)
    fire(1, 1)

    def step2(k, carry):
        for b in range(2):
            g = 2 * k + b
            finish(g, b)

            @pl.when(g + 2 < NCH)
            def _():
                fire(g + 2, b)
        return carry
    lax.fori_loop(0, NCH // 2, step2, 0)
    if NCH % 2:
        finish(NCH - 1, 0)
    plsc.subcore_barrier()

    _wb(acc_sh, out_h, c, sbase, s)


NR = 80


def _sc_cnt_body(dst_h, zeros_h, out_h, dst_flat, hist2d, iota_v, cnt_sh):
    c = lax.axis_index("c")
    s = lax.axis_index("s")
    wid = s * NC + c

    for j in range(NR // 16):
        iota_v[pl.ds(j * 16, 16)] = lax.iota(jnp.int32, 16) + 16 * j
    pltpu.sync_copy(zeros_h, hist2d)

    @pl.when(s == 0)
    def _():
        pltpu.sync_copy(zeros_h, cnt_sh)
    pltpu.sync_copy(dst_h.at[wid], dst_flat)
    plsc.subcore_barrier()

    ones16 = jnp.ones((16,), jnp.float32)

    def step(i, carry):
        idx = dst_flat[pl.ds(i * 16, 16)]
        r = lax.shift_right_logical(idx, 7)
        cl = lax.bitwise_and(idx, 127)
        plsc.addupdate_scatter(hist2d, [r, cl], ones16)
        return carry
    lax.fori_loop(0, EPW // 16, step, 0)

    pltpu.sync_copy(hist2d, cnt_sh.at[iota_v], add=True)
    plsc.subcore_barrier()

    @pl.when(s < NR // 8)
    def _():
        pltpu.sync_copy(cnt_sh.at[pl.ds(s * 8, 8)],
                        out_h.at[c, pl.ds(s * 8, 8)])


_MESH = plsc.VectorSubcoreMesh(core_axis_name="c", subcore_axis_name="s")

_sc_sum = pl.kernel(
    _sc_sum_body,
    out_type=(jax.ShapeDtypeStruct((NC, N, D), jnp.float32),),
    mesh=_MESH,
    scratch_types=(
        pltpu.VMEM((EPW,), jnp.int32),
        pltpu.VMEM((NCH, C), jnp.int32),
        pltpu.VMEM((C, D), jnp.float32),
        pltpu.VMEM((C, D), jnp.float32),
        pltpu.VMEM_SHARED((N, D), jnp.float32),
        pltpu.SemaphoreType.DMA,
        pltpu.SemaphoreType.DMA,
    ),
)

_sc_cnt = pl.kernel(
    _sc_cnt_body,
    out_type=(jax.ShapeDtypeStruct((NC, NR, D), jnp.float32),),
    mesh=_MESH,
    scratch_types=(
        pltpu.VMEM((EPW,), jnp.int32),
        pltpu.VMEM((NR, D), jnp.float32),
        pltpu.VMEM((NR,), jnp.int32),
        pltpu.VMEM_SHARED((NR, D), jnp.float32),
    ),
    compiler_params=pltpu.CompilerParams(needs_layout_passes=False),
)

BLK = 400


def _tc_body(relu, sum_ref, cnt_ref, x_ref, wl_ref, wr_ref, b_ref, out_ref):
    ssum = sum_ref[0] + sum_ref[1]
    cnt = cnt_ref[0, :, 0:1] + cnt_ref[1, :, 0:1]
    aggr = ssum / jnp.maximum(cnt, 1.0)
    y = (jnp.dot(aggr, wl_ref[...], preferred_element_type=jnp.float32)
         + jnp.dot(x_ref[...], wr_ref[...], preferred_element_type=jnp.float32)
         + b_ref[...])
    out_ref[...] = jnp.maximum(y, 0.0) if relu else y


def _tc_layer(summed, cnt, x, Wl, Wr, b, relu):
    grid = (N // BLK,)
    return pl.pallas_call(
        functools.partial(_tc_body, relu),
        grid=grid,
        in_specs=[
            pl.BlockSpec((NC, BLK, D), lambda i: (0, i, 0)),
            pl.BlockSpec((NC, BLK, 1), lambda i: (0, i, 0)),
            pl.BlockSpec((BLK, D), lambda i: (i, 0)),
            pl.BlockSpec((D, D), lambda i: (0, 0)),
            pl.BlockSpec((D, D), lambda i: (0, 0)),
            pl.BlockSpec((1, D), lambda i: (0, 0)),
        ],
        out_specs=pl.BlockSpec((BLK, D), lambda i: (i, 0)),
        out_shape=jax.ShapeDtypeStruct((N, D), jnp.float32),
    )(summed, cnt, x, Wl, Wr, b.reshape(1, D))


def kernel(x, edge_index, W1l, W1r, b1, W2l, W2r, b2):
    src = edge_index[0].astype(jnp.int32).reshape(NW, EPW)
    dst = edge_index[1].astype(jnp.int32).reshape(NW, NCH, C)
    (cnt2d,) = _sc_cnt(edge_index[1].astype(jnp.int32).reshape(NW, EPW),
                       jnp.zeros((NR, D), jnp.float32))
    cnt = cnt2d.reshape(NC, NR * D)[:, :N, None]
    zrows = jnp.zeros((RPT_LAST, D), jnp.float32)
    (summed1,) = _sc_sum(x, src, dst, zrows)
    h = _tc_layer(summed1, cnt, x, W1l, W1r, b1, relu=True)
    (summed2,) = _sc_sum(h, src, dst, zrows)
    return _tc_layer(summed2, cnt, h, W2l, W2r, b2, relu=False)


# speedup vs baseline: 1.0003x; 1.0003x over previous
import functools

import jax
import jax.numpy as jnp
from jax import lax
from jax.experimental import pallas as pl
from jax.experimental.pallas import tpu as pltpu
from jax.experimental.pallas import tpu_sc as plsc

N = 10000
E = 320000
D = 128
NC, NS = 2, 16
NW = NC * NS
EPW = E // NW
C = 80
NCH = EPW // C
RPT = 624
RPT_LAST = N - RPT * (NS - 1)


def _wb(src_sh, out_h, c, sbase, s):
    def wb(sz):
        pltpu.sync_copy(src_sh.at[pl.ds(sbase, sz)],
                        out_h.at[c, pl.ds(sbase, sz)])
    pl.when(s < NS - 1)(lambda: wb(RPT))
    pl.when(s == NS - 1)(lambda: wb(RPT_LAST))


def _sc_sum_body(x_h, src_h, dst_h, zeros_h, out_h,
                 src_all, dst_all, rows_v0, rows_v1, acc_sh,
                 sem0, sem1):
    c = lax.axis_index("c")
    s = lax.axis_index("s")
    wid = s * NC + c
    sbase = s * RPT

    def zstripe(sz):
        pltpu.sync_copy(zeros_h.at[pl.ds(0, sz)],
                        acc_sh.at[pl.ds(sbase, sz)])
    pl.when(s < NS - 1)(lambda: zstripe(RPT))
    pl.when(s == NS - 1)(lambda: zstripe(RPT_LAST))
    pltpu.sync_copy(src_h.at[wid], src_all)
    pltpu.sync_copy(dst_h.at[wid], dst_all)
    plsc.subcore_barrier()

    bufs = ((rows_v0, sem0), (rows_v1, sem1))

    def fire(g, b):
        rows_v, sem = bufs[b]
        idx = src_all.at[pl.ds(g * C, C)]
        pltpu.make_async_copy(x_h.at[idx], rows_v, sem).start()

    def finish(g, b):
        rows_v, sem = bufs[b]
        idx = src_all.at[pl.ds(g * C, C)]
        pltpu.make_async_copy(x_h.at[idx], rows_v, sem).wait()
        pltpu.sync_copy(rows_v, acc_sh.at[dst_all.at[g]], add=True)

    fire(0, 0)
    fire(1, 1)

    def step2(k, carry):
        for b in range(2):
            g = 2 * k + b
            finish(g, b)

            @pl.when(g + 2 < NCH)
            def _():
                fire(g + 2, b)
        return carry
    lax.fori_loop(0, NCH // 2, step2, 0)
    if NCH % 2:
        finish(NCH - 1, 0)
    plsc.subcore_barrier()

    _wb(acc_sh, out_h, c, sbase, s)


NR = 80


def _sc_cnt_body(dst_h, zeros_h, out_h, dst_flat, hist2d, iota_v, cnt_sh):
    c = lax.axis_index("c")
    s = lax.axis_index("s")
    wid = s * NC + c

    for j in range(NR // 16):
        iota_v[pl.ds(j * 16, 16)] = lax.iota(jnp.int32, 16) + 16 * j
    pltpu.sync_copy(zeros_h, hist2d)

    @pl.when(s == 0)
    def _():
        pltpu.sync_copy(zeros_h, cnt_sh)
    pltpu.sync_copy(dst_h.at[wid], dst_flat)
    plsc.subcore_barrier()

    ones16 = jnp.ones((16,), jnp.float32)

    def step(i, carry):
        idx = dst_flat[pl.ds(i * 16, 16)]
        r = lax.shift_right_logical(idx, 7)
        cl = lax.bitwise_and(idx, 127)
        plsc.addupdate_scatter(hist2d, [r, cl], ones16)
        return carry
    lax.fori_loop(0, EPW // 16, step, 0)

    pltpu.sync_copy(hist2d, cnt_sh.at[iota_v], add=True)
    plsc.subcore_barrier()

    @pl.when(s < NR // 8)
    def _():
        pltpu.sync_copy(cnt_sh.at[pl.ds(s * 8, 8)],
                        out_h.at[c, pl.ds(s * 8, 8)])


_MESH = plsc.VectorSubcoreMesh(core_axis_name="c", subcore_axis_name="s")

_sc_sum = pl.kernel(
    _sc_sum_body,
    out_type=(jax.ShapeDtypeStruct((NC, N, D), jnp.float32),),
    mesh=_MESH,
    scratch_types=(
        pltpu.VMEM((EPW,), jnp.int32),
        pltpu.VMEM((NCH, C), jnp.int32),
        pltpu.VMEM((C, D), jnp.float32),
        pltpu.VMEM((C, D), jnp.float32),
        pltpu.VMEM_SHARED((N, D), jnp.float32),
        pltpu.SemaphoreType.DMA,
        pltpu.SemaphoreType.DMA,
    ),
)

_sc_cnt = pl.kernel(
    _sc_cnt_body,
    out_type=(jax.ShapeDtypeStruct((NC, NR, D), jnp.float32),),
    mesh=_MESH,
    scratch_types=(
        pltpu.VMEM((EPW,), jnp.int32),
        pltpu.VMEM((NR, D), jnp.float32),
        pltpu.VMEM((NR,), jnp.int32),
        pltpu.VMEM_SHARED((NR, D), jnp.float32),
    ),
    compiler_params=pltpu.CompilerParams(needs_layout_passes=False),
)

BLK = 400


def _tc_body(relu, sum_ref, cnt_ref, x_ref, wl_ref, wr_ref, b_ref, out_ref):
    ssum = sum_ref[0] + sum_ref[1]
    cnt = cnt_ref[0, :, 0:1] + cnt_ref[1, :, 0:1]
    aggr = ssum / jnp.maximum(cnt, 1.0)
    y = (jnp.dot(aggr, wl_ref[...], preferred_element_type=jnp.float32)
         + jnp.dot(x_ref[...], wr_ref[...], preferred_element_type=jnp.float32)
         + b_ref[...])
    out_ref[...] = jnp.maximum(y, 0.0) if relu else y


def _tc_layer(summed, cnt, x, Wl, Wr, b, relu):
    grid = (N // BLK,)
    return pl.pallas_call(
        functools.partial(_tc_body, relu),
        grid=grid,
        in_specs=[
            pl.BlockSpec((NC, BLK, D), lambda i: (0, i, 0)),
            pl.BlockSpec((NC, BLK, 1), lambda i: (0, i, 0)),
            pl.BlockSpec((BLK, D), lambda i: (i, 0)),
            pl.BlockSpec((D, D), lambda i: (0, 0)),
            pl.BlockSpec((D, D), lambda i: (0, 0)),
            pl.BlockSpec((1, D), lambda i: (0, 0)),
        ],
        out_specs=pl.BlockSpec((BLK, D), lambda i: (i, 0)),
        out_shape=jax.ShapeDtypeStruct((N, D), jnp.float32),
    )(summed, cnt, x, Wl, Wr, b.reshape(1, D))


def kernel(x, edge_index, W1l, W1r, b1, W2l, W2r, b2):
    src = edge_index[0].astype(jnp.int32).reshape(NW, EPW)
    dst = edge_index[1].astype(jnp.int32).reshape(NW, NCH, C)
    (cnt2d,) = _sc_cnt(edge_index[1].astype(jnp.int32).reshape(NW, EPW),
                       jnp.zeros((NR, D), jnp.float32))
    cnt = cnt2d.reshape(NC, NR * D)[:, :N, None]
    zrows = jnp.zeros((RPT_LAST, D), jnp.float32)
    (summed1,) = _sc_sum(x, src, dst, zrows)
    h = _tc_layer(summed1, cnt, x, W1l, W1r, b1, relu=True)
    (summed2,) = _sc_sum(h, src, dst, zrows)
    return _tc_layer(summed2, cnt, h, W2l, W2r, b2, relu=False)


# speedup vs baseline: 1.0118x; 1.0115x over previous
import functools

import jax
import jax.numpy as jnp
from jax import lax
from jax.experimental import pallas as pl
from jax.experimental.pallas import tpu as pltpu
from jax.experimental.pallas import tpu_sc as plsc

N = 10000
E = 320000
D = 128
NC, NS = 2, 16
NW = NC * NS
EPW = E // NW
C = 80
NCH = EPW // C
RPT = 624
RPT_LAST = N - RPT * (NS - 1)


def _wb(src_sh, out_h, c, sbase, s):
    def wb(sz):
        pltpu.sync_copy(src_sh.at[pl.ds(sbase, sz)],
                        out_h.at[c, pl.ds(sbase, sz)])
    pl.when(s < NS - 1)(lambda: wb(RPT))
    pl.when(s == NS - 1)(lambda: wb(RPT_LAST))


def _sc_sum_body(x_h, src_h, dst_h, zeros_h, out_h,
                 src_all, dst_all, rows_v0, rows_v1, acc_sh,
                 sem0, sem1, sem2):
    c = lax.axis_index("c")
    s = lax.axis_index("s")
    wid = s * NC + c
    sbase = s * RPT

    def zdesc(sz):
        return pltpu.make_async_copy(zeros_h.at[pl.ds(0, sz)],
                                     acc_sh.at[pl.ds(sbase, sz)], sem2)
    pl.when(s < NS - 1)(lambda: zdesc(RPT).start())
    pl.when(s == NS - 1)(lambda: zdesc(RPT_LAST).start())
    pltpu.make_async_copy(src_h.at[wid], src_all, sem0).start()
    pltpu.make_async_copy(dst_h.at[wid], dst_all, sem1).start()
    pl.when(s < NS - 1)(lambda: zdesc(RPT).wait())
    pl.when(s == NS - 1)(lambda: zdesc(RPT_LAST).wait())
    pltpu.make_async_copy(src_h.at[wid], src_all, sem0).wait()
    pltpu.make_async_copy(dst_h.at[wid], dst_all, sem1).wait()
    plsc.subcore_barrier()

    bufs = ((rows_v0, sem0), (rows_v1, sem1))

    def fire(g, b):
        rows_v, sem = bufs[b]
        idx = src_all.at[pl.ds(g * C, C)]
        pltpu.make_async_copy(x_h.at[idx], rows_v, sem).start()

    def finish(g, b):
        rows_v, sem = bufs[b]
        idx = src_all.at[pl.ds(g * C, C)]
        pltpu.make_async_copy(x_h.at[idx], rows_v, sem).wait()
        pltpu.sync_copy(rows_v, acc_sh.at[dst_all.at[g]], add=True)

    fire(0, 0)
    fire(1, 1)

    def step2(k, carry):
        for b in range(2):
            g = 2 * k + b
            finish(g, b)

            @pl.when(g + 2 < NCH)
            def _():
                fire(g + 2, b)
        return carry
    lax.fori_loop(0, NCH // 2, step2, 0)
    if NCH % 2:
        finish(NCH - 1, 0)
    plsc.subcore_barrier()

    _wb(acc_sh, out_h, c, sbase, s)


NR = 80


def _sc_cnt_body(dst_h, zeros_h, out_h, dst_flat, hist2d, iota_v, cnt_sh):
    c = lax.axis_index("c")
    s = lax.axis_index("s")
    wid = s * NC + c

    for j in range(NR // 16):
        iota_v[pl.ds(j * 16, 16)] = lax.iota(jnp.int32, 16) + 16 * j
    pltpu.sync_copy(zeros_h, hist2d)

    @pl.when(s == 0)
    def _():
        pltpu.sync_copy(zeros_h, cnt_sh)
    pltpu.sync_copy(dst_h.at[wid], dst_flat)
    plsc.subcore_barrier()

    ones16 = jnp.ones((16,), jnp.float32)

    def step(i, carry):
        idx = dst_flat[pl.ds(i * 16, 16)]
        r = lax.shift_right_logical(idx, 7)
        cl = lax.bitwise_and(idx, 127)
        plsc.addupdate_scatter(hist2d, [r, cl], ones16)
        return carry
    lax.fori_loop(0, EPW // 16, step, 0)

    pltpu.sync_copy(hist2d, cnt_sh.at[iota_v], add=True)
    plsc.subcore_barrier()

    @pl.when(s < NR // 8)
    def _():
        pltpu.sync_copy(cnt_sh.at[pl.ds(s * 8, 8)],
                        out_h.at[c, pl.ds(s * 8, 8)])


_MESH = plsc.VectorSubcoreMesh(core_axis_name="c", subcore_axis_name="s")

_sc_sum = pl.kernel(
    _sc_sum_body,
    out_type=(jax.ShapeDtypeStruct((NC, N, D), jnp.float32),),
    mesh=_MESH,
    scratch_types=(
        pltpu.VMEM((EPW,), jnp.int32),
        pltpu.VMEM((NCH, C), jnp.int32),
        pltpu.VMEM((C, D), jnp.float32),
        pltpu.VMEM((C, D), jnp.float32),
        pltpu.VMEM_SHARED((N, D), jnp.float32),
        pltpu.SemaphoreType.DMA,
        pltpu.SemaphoreType.DMA,
        pltpu.SemaphoreType.DMA,
    ),
)

_sc_cnt = pl.kernel(
    _sc_cnt_body,
    out_type=(jax.ShapeDtypeStruct((NC, NR, D), jnp.float32),),
    mesh=_MESH,
    scratch_types=(
        pltpu.VMEM((EPW,), jnp.int32),
        pltpu.VMEM((NR, D), jnp.float32),
        pltpu.VMEM((NR,), jnp.int32),
        pltpu.VMEM_SHARED((NR, D), jnp.float32),
    ),
    compiler_params=pltpu.CompilerParams(needs_layout_passes=False),
)

BLK = 400


def _tc_body(relu, sum_ref, cnt_ref, x_ref, wl_ref, wr_ref, b_ref, out_ref):
    ssum = sum_ref[0] + sum_ref[1]
    cnt = cnt_ref[0, :, 0:1] + cnt_ref[1, :, 0:1]
    aggr = ssum / jnp.maximum(cnt, 1.0)
    y = (jnp.dot(aggr, wl_ref[...], preferred_element_type=jnp.float32)
         + jnp.dot(x_ref[...], wr_ref[...], preferred_element_type=jnp.float32)
         + b_ref[...])
    out_ref[...] = jnp.maximum(y, 0.0) if relu else y


def _tc_layer(summed, cnt, x, Wl, Wr, b, relu):
    grid = (N // BLK,)
    return pl.pallas_call(
        functools.partial(_tc_body, relu),
        grid=grid,
        in_specs=[
            pl.BlockSpec((NC, BLK, D), lambda i: (0, i, 0)),
            pl.BlockSpec((NC, BLK, 1), lambda i: (0, i, 0)),
            pl.BlockSpec((BLK, D), lambda i: (i, 0)),
            pl.BlockSpec((D, D), lambda i: (0, 0)),
            pl.BlockSpec((D, D), lambda i: (0, 0)),
            pl.BlockSpec((1, D), lambda i: (0, 0)),
        ],
        out_specs=pl.BlockSpec((BLK, D), lambda i: (i, 0)),
        out_shape=jax.ShapeDtypeStruct((N, D), jnp.float32),
    )(summed, cnt, x, Wl, Wr, b.reshape(1, D))


def kernel(x, edge_index, W1l, W1r, b1, W2l, W2r, b2):
    src = edge_index[0].astype(jnp.int32).reshape(NW, EPW)
    dst = edge_index[1].astype(jnp.int32).reshape(NW, NCH, C)
    (cnt2d,) = _sc_cnt(edge_index[1].astype(jnp.int32).reshape(NW, EPW),
                       jnp.zeros((NR, D), jnp.float32))
    cnt = cnt2d.reshape(NC, NR * D)[:, :N, None]
    zrows = jnp.zeros((RPT_LAST, D), jnp.float32)
    (summed1,) = _sc_sum(x, src, dst, zrows)
    h = _tc_layer(summed1, cnt, x, W1l, W1r, b1, relu=True)
    (summed2,) = _sc_sum(h, src, dst, zrows)
    return _tc_layer(summed2, cnt, h, W2l, W2r, b2, relu=False)
